# Initial kernel scaffold; baseline (speedup 1.0000x reference)
import jax
import jax.numpy as jnp
from jax.experimental import pallas as pl


def kernel(labels, logits, candidate_ids):
    raise NotImplementedError("write your pallas kernel here")


# speedup vs baseline: 1.7509x; 1.7509x over previous
import functools

import jax
import jax.numpy as jnp
import numpy as np
from jax.experimental import pallas as pl

_MIN_FLOAT = np.finfo(np.float32).min / 100.0
_BR = 256


def _body(cid_ref, labels_ref, logits_ref, out_ref):
    lab = labels_ref[...]
    cid = cid_ref[...]
    pcid = jnp.sum(lab * cid, axis=1, keepdims=True)
    eq = (cid == pcid).astype(jnp.float32)
    out_ref[...] = logits_ref[...] + (eq - lab) * _MIN_FLOAT


@functools.partial(jax.jit, static_argnames=())
def kernel(labels, logits, candidate_ids):
    b, c = logits.shape
    cidf = candidate_ids.astype(jnp.float32).reshape(1, c)
    grid = b // _BR
    return pl.pallas_call(
        _body,
        grid=(grid,),
        in_specs=[
            pl.BlockSpec((1, c), lambda i: (0, 0)),
            pl.BlockSpec((_BR, c), lambda i: (i, 0)),
            pl.BlockSpec((_BR, c), lambda i: (i, 0)),
        ],
        out_specs=pl.BlockSpec((_BR, c), lambda i: (i, 0)),
        out_shape=jax.ShapeDtypeStruct((b, c), jnp.float32),
    )(cidf, labels, logits)
